# baseline (speedup 1.0000x reference)
import functools

import jax
import jax.numpy as jnp
from jax import lax
from jax.experimental import pallas as pl
from jax.experimental.pallas import tpu as pltpu
from jax.experimental.pallas import tpu_sc as plsc

_N = 10000
_E = 320000
_IN_DIM = 128
_HID = 64

_NC = 2
_NS = 16
_NW = _NC * _NS
_CHUNK = 128
_CPW = 80
_CPWA = _CPW + 2
_EPW = _CPW * _CHUNK
_E_PAD = _NW * _EPW
_N_ACC = 10240
_ZROWS = _N_ACC // _NS
_RPS = _N // _NS

_MESH = plsc.VectorSubcoreMesh(core_axis_name="c", subcore_axis_name="s")


@functools.partial(
    pl.kernel,
    out_type=jax.ShapeDtypeStruct((_NC, _N_ACC, _HID), jnp.float32),
    mesh=_MESH,
    scratch_types=[
        pltpu.VMEM((_CPWA, _CHUNK), jnp.int32),
        pltpu.VMEM((_CPWA, _CHUNK), jnp.int32),
        pltpu.VMEM((_CHUNK, _HID), jnp.float32),
        pltpu.VMEM((_CHUNK, _HID), jnp.float32),
        pltpu.VMEM((_ZROWS, _HID), jnp.float32),
        pltpu.VMEM_SHARED((_N_ACC, _HID), jnp.float32),
        pltpu.SemaphoreType.DMA,
        pltpu.SemaphoreType.DMA,
    ],
    compiler_params=pltpu.CompilerParams(use_tc_tiling_on_sc=False),
)
def _sc_scatter_add(h_hbm, src_hbm, dst_hbm, z_hbm, out_hbm,
                    src_v, dst_v, rows_a, rows_b, stage_v, acc, sem_a, sem_b):
    c = lax.axis_index("c")
    s = lax.axis_index("s")
    wid = s * _NC + c
    pltpu.sync_copy(src_hbm.at[wid], src_v)
    pltpu.sync_copy(dst_hbm.at[wid], dst_v)
    pltpu.sync_copy(z_hbm, stage_v)
    pltpu.sync_copy(stage_v, acc.at[pl.ds(s * _ZROWS, _ZROWS)])
    plsc.subcore_barrier()

    pltpu.async_copy(h_hbm.at[src_v.at[0]], rows_a, sem_a)
    pltpu.async_copy(h_hbm.at[src_v.at[1]], rows_b, sem_b)

    def body(jj, carry):
        j = jj * 2
        pltpu.make_async_copy(h_hbm.at[src_v.at[j]], rows_a, sem_a).wait()
        pltpu.sync_copy(rows_a, acc.at[dst_v.at[j]], add=True)
        pltpu.async_copy(h_hbm.at[src_v.at[j + 2]], rows_a, sem_a)
        pltpu.make_async_copy(h_hbm.at[src_v.at[j]], rows_b, sem_b).wait()
        pltpu.sync_copy(rows_b, acc.at[dst_v.at[j + 1]], add=True)
        pltpu.async_copy(h_hbm.at[src_v.at[j + 3]], rows_b, sem_b)
        return carry

    lax.fori_loop(0, _CPW // 2, body, 0)
    pltpu.make_async_copy(h_hbm.at[src_v.at[0]], rows_a, sem_a).wait()
    pltpu.make_async_copy(h_hbm.at[src_v.at[0]], rows_b, sem_b).wait()
    plsc.subcore_barrier()
    pltpu.sync_copy(acc.at[pl.ds(s * _ZROWS, _ZROWS)], stage_v)
    pltpu.sync_copy(stage_v, out_hbm.at[c, pl.ds(s * _ZROWS, _ZROWS)])


@functools.partial(
    pl.kernel,
    out_type=jax.ShapeDtypeStruct((_NW, _CPW, _CHUNK), jnp.float32),
    mesh=_MESH,
    scratch_types=[
        pltpu.VMEM((_CPWA, _CHUNK), jnp.int32),
        pltpu.VMEM((_CPWA, _CHUNK), jnp.int32),
        pltpu.VMEM((2, _CHUNK, _HID), jnp.float32),
        pltpu.VMEM((2, _CHUNK, _HID), jnp.float32),
        pltpu.VMEM((_CPW, _CHUNK), jnp.float32),
        pltpu.SemaphoreType.DMA,
        pltpu.SemaphoreType.DMA,
        pltpu.SemaphoreType.DMA,
        pltpu.SemaphoreType.DMA,
    ],
    compiler_params=pltpu.CompilerParams(use_tc_tiling_on_sc=False,
                                         needs_layout_passes=False),
)
def _sc_edge_dot(hs_hbm, src_hbm, dst_hbm, out_hbm,
                 src_v, dst_v, a_v, b_v, o_v, sem_a0, sem_b0, sem_a1, sem_b1):
    c = lax.axis_index("c")
    s = lax.axis_index("s")
    wid = s * _NC + c
    pltpu.sync_copy(src_hbm.at[wid], src_v)
    pltpu.sync_copy(dst_hbm.at[wid], dst_v)

    sems = (sem_a0, sem_b0, sem_a1, sem_b1)
    pltpu.async_copy(hs_hbm.at[src_v.at[0]], a_v.at[0], sem_a0)
    pltpu.async_copy(hs_hbm.at[dst_v.at[0]], b_v.at[0], sem_b0)
    pltpu.async_copy(hs_hbm.at[src_v.at[1]], a_v.at[1], sem_a1)
    pltpu.async_copy(hs_hbm.at[dst_v.at[1]], b_v.at[1], sem_b1)

    def dot_chunk(j, slot):
        av_ref = a_v.at[slot]
        bv_ref = b_v.at[slot]
        pltpu.make_async_copy(hs_hbm.at[src_v.at[j]], av_ref, sems[2 * slot]).wait()
        pltpu.make_async_copy(hs_hbm.at[src_v.at[j]], bv_ref, sems[2 * slot + 1]).wait()

        def grp_body(g, gcarry):
            e16 = lax.iota(jnp.int32, 16) + g * 16
            accs = [jnp.zeros((16,), jnp.float32) for _ in range(4)]
            for d in range(_HID):
                dv = jnp.full((16,), d, jnp.int32)
                av = plsc.load_gather(av_ref, [e16, dv])
                bv = plsc.load_gather(bv_ref, [e16, dv])
                accs[d % 4] = accs[d % 4] + av * bv
            o_v[j, pl.ds(g * 16, 16)] = (accs[0] + accs[1]) + (accs[2] + accs[3])
            return gcarry

        lax.fori_loop(0, _CHUNK // 16, grp_body, 0)
        pltpu.async_copy(hs_hbm.at[src_v.at[j + 2]], av_ref, sems[2 * slot])
        pltpu.async_copy(hs_hbm.at[dst_v.at[j + 2]], bv_ref, sems[2 * slot + 1])

    def chunk_body(jj, carry):
        j = jj * 2
        dot_chunk(j, 0)
        dot_chunk(j + 1, 1)
        return carry

    lax.fori_loop(0, _CPW // 2, chunk_body, 0)
    for sem in sems:
        pltpu.make_async_copy(hs_hbm.at[src_v.at[0]], a_v.at[0], sem).wait()
    pltpu.sync_copy(o_v, out_hbm.at[wid])


_BLK = 1000


def _tc_linear(x, w, b, relu):
    n, k = x.shape
    m = w.shape[1]

    def body(x_ref, w_ref, b_ref, o_ref):
        y = lax.dot_general(x_ref[...], w_ref[...], (((1,), (0,)), ((), ())),
                            preferred_element_type=jnp.float32) + b_ref[...]
        o_ref[...] = jnp.maximum(y, 0.0) if relu else y

    return pl.pallas_call(
        body,
        grid=(n // _BLK,),
        in_specs=[
            pl.BlockSpec((_BLK, k), lambda i: (i, 0)),
            pl.BlockSpec((k, m), lambda i: (0, 0)),
            pl.BlockSpec((1, m), lambda i: (0, 0)),
        ],
        out_specs=pl.BlockSpec((_BLK, m), lambda i: (i, 0)),
        out_shape=jax.ShapeDtypeStruct((n, m), jnp.float32),
    )(x, w, b.reshape(1, m))


def _tc_gin_mlp(h, agg, w1, b1, w2, b2, relu_out):
    n = h.shape[0]
    m = w2.shape[1]

    def body(h_ref, a_ref, w1_ref, b1_ref, w2_ref, b2_ref, o_ref):
        z = h_ref[...] + a_ref[0] + a_ref[1]
        t = lax.dot_general(z, w1_ref[...], (((1,), (0,)), ((), ())),
                            preferred_element_type=jnp.float32) + b1_ref[...]
        t = jnp.maximum(t, 0.0)
        y = lax.dot_general(t, w2_ref[...], (((1,), (0,)), ((), ())),
                            preferred_element_type=jnp.float32) + b2_ref[...]
        o_ref[...] = jnp.maximum(y, 0.0) if relu_out else y

    return pl.pallas_call(
        body,
        grid=(n // _BLK,),
        in_specs=[
            pl.BlockSpec((_BLK, _HID), lambda i: (i, 0)),
            pl.BlockSpec((_NC, _BLK, _HID), lambda i: (0, i, 0)),
            pl.BlockSpec((_HID, _HID), lambda i: (0, 0)),
            pl.BlockSpec((1, _HID), lambda i: (0, 0)),
            pl.BlockSpec((_HID, m), lambda i: (0, 0)),
            pl.BlockSpec((1, m), lambda i: (0, 0)),
        ],
        out_specs=pl.BlockSpec((_BLK, m), lambda i: (i, 0)),
        out_shape=jax.ShapeDtypeStruct((n, m), jnp.float32),
    )(h, agg, w1, b1.reshape(1, _HID), w2, b2.reshape(1, m))


def kernel(x, edge_index, W_lin, b_lin, enc1_W1, enc1_b1, enc1_W2, enc1_b2,
           enc2_W1, enc2_b1, enc2_W2, enc2_b2, dec1_W1, dec1_b1, dec1_W2,
           dec1_b2, dec2_W1, dec2_b1, dec2_W2, dec2_b2, Ws, bs):
    src = edge_index[0]
    dst = edge_index[1]
    pad = _E_PAD - _E
    look = jnp.zeros((_NW, 2, _CHUNK), jnp.int32)
    srcp = jnp.concatenate([
        jnp.concatenate([src, jnp.zeros((pad,), jnp.int32)]
                        ).reshape(_NW, _CPW, _CHUNK), look], axis=1)
    dst_sc = jnp.concatenate([
        jnp.concatenate([dst, jnp.full((pad,), _N, jnp.int32)]
                        ).reshape(_NW, _CPW, _CHUNK),
        jnp.full((_NW, 2, _CHUNK), _N, jnp.int32)], axis=1)
    dst_g = jnp.concatenate([
        jnp.concatenate([dst, jnp.zeros((pad,), jnp.int32)]
                        ).reshape(_NW, _CPW, _CHUNK), look], axis=1)
    zeros_blk = jnp.zeros((_ZROWS, _HID), jnp.float32)

    def gin(h, w1, b1, w2, b2, relu_out):
        agg = _sc_scatter_add(h, srcp, dst_sc, zeros_blk)
        return _tc_gin_mlp(h, agg, w1, b1, w2, b2, relu_out)

    h0 = _tc_linear(x, W_lin, b_lin, relu=False)
    h1 = gin(h0, enc1_W1, enc1_b1, enc1_W2, enc1_b2, True)
    emb = gin(h1, enc2_W1, enc2_b1, enc2_W2, enc2_b2, False)
    a = gin(emb, dec1_W1, dec1_b1, dec1_W2, dec1_b2, True)
    x_ = gin(a, dec2_W1, dec2_b1, dec2_W2, dec2_b2, False)

    hs = _tc_linear(emb, Ws, bs, relu=True)
    s_pad = _sc_edge_dot(hs, srcp, dst_g)
    s_ = s_pad.reshape(-1)[:_E]
    return (x_, s_)


# speedup vs baseline: 1.2359x; 1.2359x over previous
import functools

import jax
import jax.numpy as jnp
from jax import lax
from jax.experimental import pallas as pl
from jax.experimental.pallas import tpu as pltpu
from jax.experimental.pallas import tpu_sc as plsc

_N = 10000
_E = 320000
_IN_DIM = 128
_HID = 64

_NC = 2
_NS = 16
_NW = _NC * _NS
_EB = 4096
_GPB = _EB // 16

_E_PAD = 327680
_EXTRA = 2 * _EB

_N_ACC = 10240

_SC_F = 4
_SC_EPR = _E_PAD // _NC
_SC_NBLK = _SC_EPR // _EB
_DT_F = 8
_DT_NR = 4
_DT_EPR = _E_PAD // _DT_NR
_DT_NBLK = _DT_EPR // _EB

_MESH = plsc.VectorSubcoreMesh(core_axis_name="c", subcore_axis_name="s")

_NB = 10
_BLK = 1000
_SC_PIECE = _BLK * _SC_F
_DT_PIECE = _BLK * _DT_F


@functools.partial(
    pl.kernel,
    out_type=jax.ShapeDtypeStruct((_NC, _NB, _NS, _SC_PIECE), jnp.float32),
    mesh=_MESH,
    scratch_types=[
        pltpu.VMEM((_N * _SC_F,), jnp.float32),
        pltpu.VMEM((_N_ACC * _SC_F,), jnp.float32),
        pltpu.VMEM((2, _EB), jnp.int32),
        pltpu.VMEM((2, _EB), jnp.int32),
        pltpu.SemaphoreType.DMA,
        pltpu.SemaphoreType.DMA,
        pltpu.SemaphoreType.DMA,
        pltpu.SemaphoreType.DMA,
        pltpu.SemaphoreType.DMA,
    ],
    compiler_params=pltpu.CompilerParams(use_tc_tiling_on_sc=False,
                                         needs_layout_passes=False),
)
def _sc_scatter_add(hsl_hbm, src_hbm, dst_hbm, out_hbm,
                    tbl_v, acc_v, src_v, dst_v,
                    sem_t, sem_s0, sem_d0, sem_s1, sem_d1):
    c = lax.axis_index("c")
    s = lax.axis_index("s")
    base = c * _SC_EPR
    ssems = (sem_s0, sem_s1)
    dsems = (sem_d0, sem_d1)

    cts = [pltpu.async_copy(hsl_hbm.at[nb, s],
                            tbl_v.at[pl.ds(nb * _SC_PIECE, _SC_PIECE)],
                            sem_t)
           for nb in range(_NB)]
    pltpu.async_copy(src_hbm.at[pl.ds(base, _EB)], src_v.at[0], sem_s0)
    pltpu.async_copy(dst_hbm.at[pl.ds(base, _EB)], dst_v.at[0], sem_d0)
    pltpu.async_copy(src_hbm.at[pl.ds(base + _EB, _EB)], src_v.at[1], sem_s1)
    pltpu.async_copy(dst_hbm.at[pl.ds(base + _EB, _EB)], dst_v.at[1], sem_d1)

    def zbody(i, carry):
        acc_v[pl.ds(i * 16, 16)] = jnp.zeros((16,), jnp.float32)
        return carry

    lax.fori_loop(0, _N_ACC * _SC_F // 16, zbody, 0)
    for ct in cts:
        ct.wait()

    def do_block(b, slot):
        sref = src_v.at[slot]
        dref = dst_v.at[slot]
        pltpu.make_async_copy(src_hbm.at[pl.ds(0, _EB)], sref,
                              ssems[slot]).wait()
        pltpu.make_async_copy(dst_hbm.at[pl.ds(0, _EB)], dref,
                              dsems[slot]).wait()

        def grp(g, carry):
            src16 = sref[pl.ds(g * 16, 16)]
            dst16 = dref[pl.ds(g * 16, 16)]
            si = src16 * _SC_F
            di = dst16 * _SC_F
            for f in range(_SC_F):
                v = plsc.load_gather(tbl_v, [si + f])
                plsc.addupdate_scatter(acc_v, [di + f], v)
            return carry

        lax.fori_loop(0, _GPB, grp, 0)
        off = base + (b + 2) * _EB
        pltpu.async_copy(src_hbm.at[pl.ds(off, _EB)], sref, ssems[slot])
        pltpu.async_copy(dst_hbm.at[pl.ds(off, _EB)], dref, dsems[slot])

    def body(bb, carry):
        do_block(bb * 2, 0)
        do_block(bb * 2 + 1, 1)
        return carry

    lax.fori_loop(0, _SC_NBLK // 2, body, 0)
    for sem in (sem_s0, sem_d0, sem_s1, sem_d1):
        pltpu.make_async_copy(src_hbm.at[pl.ds(0, _EB)], src_v.at[0],
                              sem).wait()
    for nb in range(_NB):
        pltpu.sync_copy(acc_v.at[pl.ds(nb * _SC_PIECE, _SC_PIECE)],
                        out_hbm.at[c, nb, s])


@functools.partial(
    pl.kernel,
    out_type=jax.ShapeDtypeStruct((_DT_F, _E_PAD), jnp.float32),
    mesh=_MESH,
    scratch_types=[
        pltpu.VMEM((_N * _DT_F,), jnp.float32),
        pltpu.VMEM((2, _EB), jnp.int32),
        pltpu.VMEM((2, _EB), jnp.int32),
        pltpu.VMEM((2, _EB), jnp.float32),
        pltpu.SemaphoreType.DMA,
        pltpu.SemaphoreType.DMA,
        pltpu.SemaphoreType.DMA,
        pltpu.SemaphoreType.DMA,
        pltpu.SemaphoreType.DMA,
        pltpu.SemaphoreType.DMA,
        pltpu.SemaphoreType.DMA,
    ],
    compiler_params=pltpu.CompilerParams(use_tc_tiling_on_sc=False,
                                         needs_layout_passes=False),
)
def _sc_edge_dot(hsl_hbm, src_hbm, dst_hbm, out_hbm,
                 tbl_v, src_v, dst_v, o_v,
                 sem_t, sem_s0, sem_d0, sem_s1, sem_d1, sem_o0, sem_o1):
    c = lax.axis_index("c")
    s = lax.axis_index("s")
    k = lax.rem(s, _DT_F)
    r = lax.div(s, _DT_F) * _NC + c
    base = r * _DT_EPR
    ssems = (sem_s0, sem_s1)
    dsems = (sem_d0, sem_d1)
    osems = (sem_o0, sem_o1)

    cts = [pltpu.async_copy(hsl_hbm.at[nb, k],
                            tbl_v.at[pl.ds(nb * _DT_PIECE, _DT_PIECE)],
                            sem_t)
           for nb in range(_NB)]
    pltpu.async_copy(src_hbm.at[pl.ds(base, _EB)], src_v.at[0], sem_s0)
    pltpu.async_copy(dst_hbm.at[pl.ds(base, _EB)], dst_v.at[0], sem_d0)
    pltpu.async_copy(src_hbm.at[pl.ds(base + _EB, _EB)], src_v.at[1], sem_s1)
    pltpu.async_copy(dst_hbm.at[pl.ds(base + _EB, _EB)], dst_v.at[1], sem_d1)
    for ct in cts:
        ct.wait()

    def do_block(b, slot, first):
        sref = src_v.at[slot]
        dref = dst_v.at[slot]
        oref = o_v.at[slot]
        pltpu.make_async_copy(src_hbm.at[pl.ds(0, _EB)], sref,
                              ssems[slot]).wait()
        pltpu.make_async_copy(dst_hbm.at[pl.ds(0, _EB)], dref,
                              dsems[slot]).wait()

        @pl.when(jnp.logical_not(first))
        def _():
            pltpu.make_async_copy(oref, out_hbm.at[k, pl.ds(0, _EB)],
                                  osems[slot]).wait()

        def grp(g, carry):
            src16 = sref[pl.ds(g * 16, 16)]
            dst16 = dref[pl.ds(g * 16, 16)]
            si = src16 * _DT_F
            di = dst16 * _DT_F
            acc0 = jnp.zeros((16,), jnp.float32)
            acc1 = jnp.zeros((16,), jnp.float32)
            for f in range(_DT_F):
                va = plsc.load_gather(tbl_v, [si + f])
                vb = plsc.load_gather(tbl_v, [di + f])
                if f % 2 == 0:
                    acc0 = acc0 + va * vb
                else:
                    acc1 = acc1 + va * vb
            o_v[slot, pl.ds(g * 16, 16)] = acc0 + acc1
            return carry

        lax.fori_loop(0, _GPB, grp, 0)
        pltpu.async_copy(oref, out_hbm.at[k, pl.ds(base + b * _EB, _EB)],
                         osems[slot])
        off = base + (b + 2) * _EB
        pltpu.async_copy(src_hbm.at[pl.ds(off, _EB)], sref, ssems[slot])
        pltpu.async_copy(dst_hbm.at[pl.ds(off, _EB)], dref, dsems[slot])

    def body(bb, carry):
        do_block(bb * 2, 0, bb == 0)
        do_block(bb * 2 + 1, 1, bb == 0)
        return carry

    lax.fori_loop(0, _DT_NBLK // 2, body, 0)
    for sem in (sem_s0, sem_d0, sem_s1, sem_d1):
        pltpu.make_async_copy(src_hbm.at[pl.ds(0, _EB)], src_v.at[0],
                              sem).wait()
    for slot in (0, 1):
        pltpu.make_async_copy(o_v.at[slot], out_hbm.at[0, pl.ds(0, _EB)],
                              osems[slot]).wait()


def _slice16(y):
    blk = y.shape[0]
    return jnp.transpose(y.reshape(blk, _NS, _SC_F), (1, 0, 2)).reshape(
        _NS, blk * _SC_F)


def _slice8(y):
    blk = y.shape[0]
    return jnp.transpose(y.reshape(blk, _DT_F, _DT_F), (1, 0, 2)).reshape(
        _DT_F, blk * _DT_F)


def _matmul(a, w):
    return lax.dot_general(a, w, (((1,), (0,)), ((), ())),
                           preferred_element_type=jnp.float32)


def _tc_linear(x, w, b, relu, mode):
    n, kdim = x.shape
    m = w.shape[1]

    def body(x_ref, w_ref, b_ref, *o_refs):
        y = _matmul(x_ref[...], w_ref[...]) + b_ref[...]
        if relu:
            y = jnp.maximum(y, 0.0)
        if mode == "plain16":
            o_refs[0][...] = y
            o_refs[1][0] = _slice16(y)
        else:
            o_refs[0][0] = _slice8(y)

    if mode == "plain16":
        out_shape = [jax.ShapeDtypeStruct((n, m), jnp.float32),
                     jax.ShapeDtypeStruct((_NB, _NS, _SC_PIECE), jnp.float32)]
        out_specs = [pl.BlockSpec((_BLK, m), lambda i: (i, 0)),
                     pl.BlockSpec((1, _NS, _SC_PIECE), lambda i: (i, 0, 0))]
    else:
        out_shape = [jax.ShapeDtypeStruct((_NB, _DT_F, _DT_PIECE),
                                          jnp.float32)]
        out_specs = [pl.BlockSpec((1, _DT_F, _DT_PIECE), lambda i: (i, 0, 0))]

    return pl.pallas_call(
        body,
        grid=(n // _BLK,),
        in_specs=[
            pl.BlockSpec((_BLK, kdim), lambda i: (i, 0)),
            pl.BlockSpec((kdim, m), lambda i: (0, 0)),
            pl.BlockSpec((1, m), lambda i: (0, 0)),
        ],
        out_specs=out_specs,
        out_shape=out_shape,
    )(x, w, b.reshape(1, m))


def _tc_gin_mlp(h, agg, w1, b1, w2, b2, relu_out, emit_slices):
    n = h.shape[0]
    m = w2.shape[1]

    def body(h_ref, a_ref, w1_ref, b1_ref, w2_ref, b2_ref, *o_refs):
        asum = a_ref[0, 0] + a_ref[1, 0]
        aggb = jnp.transpose(asum.reshape(_NS, _BLK, _SC_F),
                             (1, 0, 2)).reshape(_BLK, _HID)
        z = h_ref[...] + aggb
        t = jnp.maximum(_matmul(z, w1_ref[...]) + b1_ref[...], 0.0)
        y = _matmul(t, w2_ref[...]) + b2_ref[...]
        if relu_out:
            y = jnp.maximum(y, 0.0)
        o_refs[0][...] = y
        if emit_slices:
            o_refs[1][0] = _slice16(y)

    out_shape = [jax.ShapeDtypeStruct((n, m), jnp.float32)]
    out_specs = [pl.BlockSpec((_BLK, m), lambda i: (i, 0))]
    if emit_slices:
        out_shape.append(
            jax.ShapeDtypeStruct((_NB, _NS, _SC_PIECE), jnp.float32))
        out_specs.append(
            pl.BlockSpec((1, _NS, _SC_PIECE), lambda i: (i, 0, 0)))

    return pl.pallas_call(
        body,
        grid=(n // _BLK,),
        in_specs=[
            pl.BlockSpec((_BLK, _HID), lambda i: (i, 0)),
            pl.BlockSpec((_NC, 1, _NS, _SC_PIECE), lambda i: (0, i, 0, 0)),
            pl.BlockSpec((_HID, _HID), lambda i: (0, 0)),
            pl.BlockSpec((1, _HID), lambda i: (0, 0)),
            pl.BlockSpec((_HID, m), lambda i: (0, 0)),
            pl.BlockSpec((1, m), lambda i: (0, 0)),
        ],
        out_specs=out_specs,
        out_shape=out_shape,
    )(h, agg, w1, b1.reshape(1, _HID), w2, b2.reshape(1, m))


def _tc_slice_sum(parts):
    blk = 32768

    def body(p_ref, o_ref):
        o_ref[...] = jnp.sum(p_ref[...], axis=0)

    return pl.pallas_call(
        body,
        grid=(_E_PAD // blk,),
        in_specs=[pl.BlockSpec((_DT_F, blk), lambda i: (0, i))],
        out_specs=pl.BlockSpec((blk,), lambda i: (i,)),
        out_shape=jax.ShapeDtypeStruct((_E_PAD,), jnp.float32),
    )(parts)


def kernel(x, edge_index, W_lin, b_lin, enc1_W1, enc1_b1, enc1_W2, enc1_b2,
           enc2_W1, enc2_b1, enc2_W2, enc2_b2, dec1_W1, dec1_b1, dec1_W2,
           dec1_b2, dec2_W1, dec2_b1, dec2_W2, dec2_b2, Ws, bs):
    src = edge_index[0]
    dst = edge_index[1]
    pad = _E_PAD - _E
    srcp = jnp.concatenate(
        [src, jnp.zeros((pad + _EXTRA,), jnp.int32)])
    pad_rows = _N + (jnp.arange(pad + _EXTRA, dtype=jnp.int32)
                     % (_N_ACC - _N))
    dst_sc = jnp.concatenate([dst, pad_rows])
    dst_g = jnp.concatenate([dst, jnp.zeros((pad + _EXTRA,), jnp.int32)])

    def gin(h, hsl, w1, b1, w2, b2, relu_out, emit_slices):
        agg = _sc_scatter_add(hsl, srcp, dst_sc)
        return _tc_gin_mlp(h, agg, w1, b1, w2, b2, relu_out, emit_slices)

    h0, h0sl = _tc_linear(x, W_lin, b_lin, False, "plain16")
    h1, h1sl = gin(h0, h0sl, enc1_W1, enc1_b1, enc1_W2, enc1_b2, True, True)
    emb, embsl = gin(h1, h1sl, enc2_W1, enc2_b1, enc2_W2, enc2_b2, False,
                     True)
    a, asl = gin(emb, embsl, dec1_W1, dec1_b1, dec1_W2, dec1_b2, True, True)
    (x_,) = gin(a, asl, dec2_W1, dec2_b1, dec2_W2, dec2_b2, False, False)

    (hs8,) = _tc_linear(emb, Ws, bs, True, "slice8")
    parts = _sc_edge_dot(hs8, srcp, dst_g)
    s_ = _tc_slice_sum(parts)[:_E]
    return (x_, s_)


# speedup vs baseline: 2.1557x; 1.7443x over previous
import functools

import jax
import jax.numpy as jnp
from jax import lax
from jax.experimental import pallas as pl
from jax.experimental.pallas import tpu as pltpu
from jax.experimental.pallas import tpu_sc as plsc

_N = 10000
_E = 320000
_IN_DIM = 128
_HID = 64

_NC = 2
_NS = 16
_NW = _NC * _NS
_EB = 4096
_GPB = _EB // 16

_E_PAD = 327680
_EXTRA = 2 * _EB

_N_ACC = 10240

_SC_F = 4
_SC_EPR = _E_PAD // _NC
_SC_NBLK = _SC_EPR // _EB
_DT_F = 8
_DT_NR = 4
_DT_EPR = _E_PAD // _DT_NR
_DT_NBLK = _DT_EPR // _EB

_MESH = plsc.VectorSubcoreMesh(core_axis_name="c", subcore_axis_name="s")

_NB = 10
_BLK = 1000
_SC_PIECE = _BLK * _SC_F
_DT_PIECE = _BLK * _DT_F


@functools.partial(
    pl.kernel,
    out_type=jax.ShapeDtypeStruct((_NC, _NB, _NS, _SC_PIECE), jnp.float32),
    mesh=_MESH,
    scratch_types=[
        pltpu.VMEM((_N * _SC_F,), jnp.float32),
        pltpu.VMEM((_N_ACC * _SC_F,), jnp.float32),
        pltpu.VMEM((2, _EB), jnp.int32),
        pltpu.VMEM((2, _EB), jnp.int32),
        pltpu.SemaphoreType.DMA,
        pltpu.SemaphoreType.DMA,
        pltpu.SemaphoreType.DMA,
        pltpu.SemaphoreType.DMA,
        pltpu.SemaphoreType.DMA,
    ],
    compiler_params=pltpu.CompilerParams(use_tc_tiling_on_sc=False,
                                         needs_layout_passes=False),
)
def _sc_scatter_add(hsl_hbm, src_hbm, dst_hbm, out_hbm,
                    tbl_v, acc_v, src_v, dst_v,
                    sem_t, sem_s0, sem_d0, sem_s1, sem_d1):
    c = lax.axis_index("c")
    s = lax.axis_index("s")
    base = c * _SC_EPR
    ssems = (sem_s0, sem_s1)
    dsems = (sem_d0, sem_d1)

    cts = [pltpu.async_copy(hsl_hbm.at[nb, s],
                            tbl_v.at[pl.ds(nb * _SC_PIECE, _SC_PIECE)],
                            sem_t)
           for nb in range(_NB)]
    pltpu.async_copy(src_hbm.at[pl.ds(base, _EB)], src_v.at[0], sem_s0)
    pltpu.async_copy(dst_hbm.at[pl.ds(base, _EB)], dst_v.at[0], sem_d0)
    pltpu.async_copy(src_hbm.at[pl.ds(base + _EB, _EB)], src_v.at[1], sem_s1)
    pltpu.async_copy(dst_hbm.at[pl.ds(base + _EB, _EB)], dst_v.at[1], sem_d1)

    @plsc.parallel_loop(0, _N_ACC * _SC_F // 16, unroll=8)
    def _zero(i):
        acc_v[pl.ds(i * 16, 16)] = jnp.zeros((16,), jnp.float32)

    for ct in cts:
        ct.wait()

    def do_block(b, slot):
        sref = src_v.at[slot]
        dref = dst_v.at[slot]
        pltpu.make_async_copy(src_hbm.at[pl.ds(0, _EB)], sref,
                              ssems[slot]).wait()
        pltpu.make_async_copy(dst_hbm.at[pl.ds(0, _EB)], dref,
                              dsems[slot]).wait()

        @plsc.parallel_loop(0, _GPB, unroll=4)
        def _grp(g):
            src16 = sref[pl.ds(g * 16, 16)]
            dst16 = dref[pl.ds(g * 16, 16)]
            si = src16 * _SC_F
            di = dst16 * _SC_F
            for f in range(_SC_F):
                v = plsc.load_gather(tbl_v, [si + f])
                plsc.addupdate_scatter(acc_v, [di + f], v)

        off = base + (b + 2) * _EB
        pltpu.async_copy(src_hbm.at[pl.ds(off, _EB)], sref, ssems[slot])
        pltpu.async_copy(dst_hbm.at[pl.ds(off, _EB)], dref, dsems[slot])

    def body(bb, carry):
        do_block(bb * 2, 0)
        do_block(bb * 2 + 1, 1)
        return carry

    lax.fori_loop(0, _SC_NBLK // 2, body, 0)
    for sem in (sem_s0, sem_d0, sem_s1, sem_d1):
        pltpu.make_async_copy(src_hbm.at[pl.ds(0, _EB)], src_v.at[0],
                              sem).wait()
    for nb in range(_NB):
        pltpu.sync_copy(acc_v.at[pl.ds(nb * _SC_PIECE, _SC_PIECE)],
                        out_hbm.at[c, nb, s])


@functools.partial(
    pl.kernel,
    out_type=jax.ShapeDtypeStruct((_DT_F, _E_PAD), jnp.float32),
    mesh=_MESH,
    scratch_types=[
        pltpu.VMEM((_N * _DT_F,), jnp.float32),
        pltpu.VMEM((2, _EB), jnp.int32),
        pltpu.VMEM((2, _EB), jnp.int32),
        pltpu.VMEM((2, _EB), jnp.float32),
        pltpu.SemaphoreType.DMA,
        pltpu.SemaphoreType.DMA,
        pltpu.SemaphoreType.DMA,
        pltpu.SemaphoreType.DMA,
        pltpu.SemaphoreType.DMA,
        pltpu.SemaphoreType.DMA,
        pltpu.SemaphoreType.DMA,
    ],
    compiler_params=pltpu.CompilerParams(use_tc_tiling_on_sc=False,
                                         needs_layout_passes=False),
)
def _sc_edge_dot(hsl_hbm, src_hbm, dst_hbm, out_hbm,
                 tbl_v, src_v, dst_v, o_v,
                 sem_t, sem_s0, sem_d0, sem_s1, sem_d1, sem_o0, sem_o1):
    c = lax.axis_index("c")
    s = lax.axis_index("s")
    k = lax.rem(s, _DT_F)
    r = lax.div(s, _DT_F) * _NC + c
    base = r * _DT_EPR
    ssems = (sem_s0, sem_s1)
    dsems = (sem_d0, sem_d1)
    osems = (sem_o0, sem_o1)

    cts = [pltpu.async_copy(hsl_hbm.at[nb, k],
                            tbl_v.at[pl.ds(nb * _DT_PIECE, _DT_PIECE)],
                            sem_t)
           for nb in range(_NB)]
    pltpu.async_copy(src_hbm.at[pl.ds(base, _EB)], src_v.at[0], sem_s0)
    pltpu.async_copy(dst_hbm.at[pl.ds(base, _EB)], dst_v.at[0], sem_d0)
    pltpu.async_copy(src_hbm.at[pl.ds(base + _EB, _EB)], src_v.at[1], sem_s1)
    pltpu.async_copy(dst_hbm.at[pl.ds(base + _EB, _EB)], dst_v.at[1], sem_d1)
    for ct in cts:
        ct.wait()

    def do_block(b, slot, first):
        sref = src_v.at[slot]
        dref = dst_v.at[slot]
        oref = o_v.at[slot]
        pltpu.make_async_copy(src_hbm.at[pl.ds(0, _EB)], sref,
                              ssems[slot]).wait()
        pltpu.make_async_copy(dst_hbm.at[pl.ds(0, _EB)], dref,
                              dsems[slot]).wait()

        @pl.when(jnp.logical_not(first))
        def _():
            pltpu.make_async_copy(oref, out_hbm.at[k, pl.ds(0, _EB)],
                                  osems[slot]).wait()

        @plsc.parallel_loop(0, _GPB, unroll=4)
        def _grp(g):
            src16 = sref[pl.ds(g * 16, 16)]
            dst16 = dref[pl.ds(g * 16, 16)]
            si = src16 * _DT_F
            di = dst16 * _DT_F
            acc0 = jnp.zeros((16,), jnp.float32)
            acc1 = jnp.zeros((16,), jnp.float32)
            for f in range(_DT_F):
                va = plsc.load_gather(tbl_v, [si + f])
                vb = plsc.load_gather(tbl_v, [di + f])
                if f % 2 == 0:
                    acc0 = acc0 + va * vb
                else:
                    acc1 = acc1 + va * vb
            o_v[slot, pl.ds(g * 16, 16)] = acc0 + acc1

        pltpu.async_copy(oref, out_hbm.at[k, pl.ds(base + b * _EB, _EB)],
                         osems[slot])
        off = base + (b + 2) * _EB
        pltpu.async_copy(src_hbm.at[pl.ds(off, _EB)], sref, ssems[slot])
        pltpu.async_copy(dst_hbm.at[pl.ds(off, _EB)], dref, dsems[slot])

    def body(bb, carry):
        do_block(bb * 2, 0, bb == 0)
        do_block(bb * 2 + 1, 1, bb == 0)
        return carry

    lax.fori_loop(0, _DT_NBLK // 2, body, 0)
    for sem in (sem_s0, sem_d0, sem_s1, sem_d1):
        pltpu.make_async_copy(src_hbm.at[pl.ds(0, _EB)], src_v.at[0],
                              sem).wait()
    for slot in (0, 1):
        pltpu.make_async_copy(o_v.at[slot], out_hbm.at[0, pl.ds(0, _EB)],
                              osems[slot]).wait()


def _slice16(y):
    blk = y.shape[0]
    return jnp.transpose(y.reshape(blk, _NS, _SC_F), (1, 0, 2)).reshape(
        _NS, blk * _SC_F)


def _slice8(y):
    blk = y.shape[0]
    return jnp.transpose(y.reshape(blk, _DT_F, _DT_F), (1, 0, 2)).reshape(
        _DT_F, blk * _DT_F)


def _matmul(a, w):
    return lax.dot_general(a, w, (((1,), (0,)), ((), ())),
                           preferred_element_type=jnp.float32)


def _tc_linear(x, w, b, relu, mode):
    n, kdim = x.shape
    m = w.shape[1]

    def body(x_ref, w_ref, b_ref, *o_refs):
        y = _matmul(x_ref[...], w_ref[...]) + b_ref[...]
        if relu:
            y = jnp.maximum(y, 0.0)
        if mode == "plain16":
            o_refs[0][...] = y
            o_refs[1][0] = _slice16(y)
        else:
            o_refs[0][0] = _slice8(y)

    if mode == "plain16":
        out_shape = [jax.ShapeDtypeStruct((n, m), jnp.float32),
                     jax.ShapeDtypeStruct((_NB, _NS, _SC_PIECE), jnp.float32)]
        out_specs = [pl.BlockSpec((_BLK, m), lambda i: (i, 0)),
                     pl.BlockSpec((1, _NS, _SC_PIECE), lambda i: (i, 0, 0))]
    else:
        out_shape = [jax.ShapeDtypeStruct((_NB, _DT_F, _DT_PIECE),
                                          jnp.float32)]
        out_specs = [pl.BlockSpec((1, _DT_F, _DT_PIECE), lambda i: (i, 0, 0))]

    return pl.pallas_call(
        body,
        grid=(n // _BLK,),
        in_specs=[
            pl.BlockSpec((_BLK, kdim), lambda i: (i, 0)),
            pl.BlockSpec((kdim, m), lambda i: (0, 0)),
            pl.BlockSpec((1, m), lambda i: (0, 0)),
        ],
        out_specs=out_specs,
        out_shape=out_shape,
    )(x, w, b.reshape(1, m))


def _tc_gin_mlp(h, agg, w1, b1, w2, b2, relu_out, emit_slices):
    n = h.shape[0]
    m = w2.shape[1]

    def body(h_ref, a_ref, w1_ref, b1_ref, w2_ref, b2_ref, *o_refs):
        asum = a_ref[0, 0] + a_ref[1, 0]
        aggb = jnp.transpose(asum.reshape(_NS, _BLK, _SC_F),
                             (1, 0, 2)).reshape(_BLK, _HID)
        z = h_ref[...] + aggb
        t = jnp.maximum(_matmul(z, w1_ref[...]) + b1_ref[...], 0.0)
        y = _matmul(t, w2_ref[...]) + b2_ref[...]
        if relu_out:
            y = jnp.maximum(y, 0.0)
        o_refs[0][...] = y
        if emit_slices:
            o_refs[1][0] = _slice16(y)

    out_shape = [jax.ShapeDtypeStruct((n, m), jnp.float32)]
    out_specs = [pl.BlockSpec((_BLK, m), lambda i: (i, 0))]
    if emit_slices:
        out_shape.append(
            jax.ShapeDtypeStruct((_NB, _NS, _SC_PIECE), jnp.float32))
        out_specs.append(
            pl.BlockSpec((1, _NS, _SC_PIECE), lambda i: (i, 0, 0)))

    return pl.pallas_call(
        body,
        grid=(n // _BLK,),
        in_specs=[
            pl.BlockSpec((_BLK, _HID), lambda i: (i, 0)),
            pl.BlockSpec((_NC, 1, _NS, _SC_PIECE), lambda i: (0, i, 0, 0)),
            pl.BlockSpec((_HID, _HID), lambda i: (0, 0)),
            pl.BlockSpec((1, _HID), lambda i: (0, 0)),
            pl.BlockSpec((_HID, m), lambda i: (0, 0)),
            pl.BlockSpec((1, m), lambda i: (0, 0)),
        ],
        out_specs=out_specs,
        out_shape=out_shape,
    )(h, agg, w1, b1.reshape(1, _HID), w2, b2.reshape(1, m))


def _tc_slice_sum(parts):
    blk = 32768

    def body(p_ref, o_ref):
        o_ref[...] = jnp.sum(p_ref[...], axis=0)

    return pl.pallas_call(
        body,
        grid=(_E_PAD // blk,),
        in_specs=[pl.BlockSpec((_DT_F, blk), lambda i: (0, i))],
        out_specs=pl.BlockSpec((blk,), lambda i: (i,)),
        out_shape=jax.ShapeDtypeStruct((_E_PAD,), jnp.float32),
    )(parts)


def kernel(x, edge_index, W_lin, b_lin, enc1_W1, enc1_b1, enc1_W2, enc1_b2,
           enc2_W1, enc2_b1, enc2_W2, enc2_b2, dec1_W1, dec1_b1, dec1_W2,
           dec1_b2, dec2_W1, dec2_b1, dec2_W2, dec2_b2, Ws, bs):
    src = edge_index[0]
    dst = edge_index[1]
    pad = _E_PAD - _E
    srcp = jnp.concatenate(
        [src, jnp.zeros((pad + _EXTRA,), jnp.int32)])
    pad_rows = _N + (jnp.arange(pad + _EXTRA, dtype=jnp.int32)
                     % (_N_ACC - _N))
    dst_sc = jnp.concatenate([dst, pad_rows])
    dst_g = jnp.concatenate([dst, jnp.zeros((pad + _EXTRA,), jnp.int32)])

    def gin(h, hsl, w1, b1, w2, b2, relu_out, emit_slices):
        agg = _sc_scatter_add(hsl, srcp, dst_sc)
        return _tc_gin_mlp(h, agg, w1, b1, w2, b2, relu_out, emit_slices)

    h0, h0sl = _tc_linear(x, W_lin, b_lin, False, "plain16")
    h1, h1sl = gin(h0, h0sl, enc1_W1, enc1_b1, enc1_W2, enc1_b2, True, True)
    emb, embsl = gin(h1, h1sl, enc2_W1, enc2_b1, enc2_W2, enc2_b2, False,
                     True)
    a, asl = gin(emb, embsl, dec1_W1, dec1_b1, dec1_W2, dec1_b2, True, True)
    (x_,) = gin(a, asl, dec2_W1, dec2_b1, dec2_W2, dec2_b2, False, False)

    (hs8,) = _tc_linear(emb, Ws, bs, True, "slice8")
    parts = _sc_edge_dot(hs8, srcp, dst_g)
    s_ = _tc_slice_sum(parts)[:_E]
    return (x_, s_)
